# baseline (speedup 1.0000x reference)
import jax
import jax.numpy as jnp
from jax.experimental import pallas as pl
from jax.experimental.pallas import tpu as pltpu


def _add_kernel(x_ref, pe_ref, o_ref):
    o_ref[...] = x_ref[...] + pe_ref[...]


def kernel(x, pos_emb):
    B, T, C = x.shape
    x2 = x.reshape(B * T, C)
    out = pl.pallas_call(
        _add_kernel,
        grid=(B,),
        in_specs=[
            pl.BlockSpec((T, C), lambda b: (b, 0)),
            pl.BlockSpec((T, C), lambda b: (0, 0)),
        ],
        out_specs=pl.BlockSpec((T, C), lambda b: (b, 0)),
        out_shape=jax.ShapeDtypeStruct((B * T, C), x.dtype),
        compiler_params=pltpu.CompilerParams(
            dimension_semantics=("arbitrary",),
        ),
    )(x2, pos_emb)
    return out.reshape(B, T, C)
